# baseline (speedup 1.0000x reference)
import jax
import jax.numpy as jnp
from jax import lax
from jax.experimental import pallas as pl
from jax.experimental.pallas import tpu as pltpu
from jax.experimental.pallas import tpu_sc as plsc

NC = 1
NS = 16
L = 16
NW = NC * NS

B = 16384
D = 32
BPW = B // NW
BLOCKS = BPW // L


def _rsqrt_nr(x):
    i = plsc.bitcast(x, jnp.int32)
    i = jnp.int32(0x5F3759DF) - lax.shift_right_logical(i, 1)
    y = plsc.bitcast(i, jnp.float32)
    xh = x * jnp.float32(0.5)
    for _ in range(3):
        y = y * (jnp.float32(1.5) - xh * y * y)
    return y


def _sc_transe(node_hbm, rel_hbm, hidx_hbm, ridx_hbm, tidx_hbm, out_hbm,
               hiv, riv, tiv, hrows, rrows, trows,
               outv, s1, s2, s3):
    wid = lax.axis_index("s")
    base = wid * BPW

    pltpu.sync_copy(hidx_hbm.at[pl.ds(base, BPW)], hiv)
    pltpu.sync_copy(tidx_hbm.at[pl.ds(base, BPW)], tiv)
    pltpu.sync_copy(ridx_hbm.at[pl.ds(base, BPW)], riv)

    c1 = pltpu.async_copy(node_hbm.at[hiv], hrows, s1)
    c2 = pltpu.async_copy(node_hbm.at[tiv], trows, s2)
    c3 = pltpu.async_copy(rel_hbm.at[riv], rrows, s3)
    c1.wait()
    c2.wait()
    c3.wait()

    iota = lax.iota(jnp.int32, L)
    zero = jnp.zeros((L,), jnp.float32)

    def block(b, _):
        ridx = b * L + iota
        hh = tt = rr = hr = ht = rt = zero
        for d in range(D):
            cd = jnp.full((L,), d, dtype=jnp.int32)
            h = plsc.load_gather(hrows, [ridx, cd])
            t = plsc.load_gather(trows, [ridx, cd])
            r = plsc.load_gather(rrows, [ridx, cd])
            hh = hh + h * h
            tt = tt + t * t
            rr = rr + r * r
            hr = hr + h * r
            ht = ht + h * t
            rt = rt + r * t
        irh = _rsqrt_nr(jnp.maximum(hh, jnp.float32(1e-24)))
        irt = _rsqrt_nr(jnp.maximum(tt, jnp.float32(1e-24)))
        aa = hh * irh * irh
        bb = tt * irt * irt
        cross = hr * irh - ht * (irh * irt) - rt * irt
        dd = aa + bb + rr + (cross + cross)
        s = jnp.maximum(dd, jnp.float32(0.0))
        outv[pl.ds(b * L, L)] = -(s * _rsqrt_nr(jnp.maximum(s, jnp.float32(1e-30))))
        return _

    lax.fori_loop(0, BLOCKS, block, None)

    pltpu.sync_copy(outv, out_hbm.at[pl.ds(base, BPW)])


@jax.jit
def _transe_sc(node_emb, rel_emb, hidx, ridx, tidx):
    mesh = plsc.VectorSubcoreMesh(
        core_axis_name="c", subcore_axis_name="s",
        num_cores=NC, num_subcores=NS)
    f = pl.kernel(
        _sc_transe,
        out_type=jax.ShapeDtypeStruct((B,), jnp.float32),
        mesh=mesh,
        compiler_params=pltpu.CompilerParams(
            needs_layout_passes=False, use_tc_tiling_on_sc=False),
        scratch_types=[
            pltpu.VMEM((BPW,), jnp.int32),
            pltpu.VMEM((BPW,), jnp.int32),
            pltpu.VMEM((BPW,), jnp.int32),
            pltpu.VMEM((BPW, D), jnp.float32),
            pltpu.VMEM((BPW, D), jnp.float32),
            pltpu.VMEM((BPW, D), jnp.float32),
            pltpu.VMEM((BPW,), jnp.float32),
            pltpu.SemaphoreType.DMA,
            pltpu.SemaphoreType.DMA,
            pltpu.SemaphoreType.DMA,
        ],
    )
    return f(node_emb, rel_emb, hidx, ridx, tidx)


def kernel(head_index, rel_type, tail_index, node_emb, rel_emb):
    hidx = head_index.astype(jnp.int32)
    ridx = rel_type.astype(jnp.int32)
    tidx = tail_index.astype(jnp.int32)
    return _transe_sc(node_emb, rel_emb, hidx, ridx, tidx)


# speedup vs baseline: 1.0944x; 1.0944x over previous
import jax
import jax.numpy as jnp
from jax import lax
from jax.experimental import pallas as pl
from jax.experimental.pallas import tpu as pltpu
from jax.experimental.pallas import tpu_sc as plsc

NC = 2
NS = 16
L = 16
NW = NC * NS

B = 16384
D = 32
BPW = B // NW
BLOCKS = BPW // L


def _rsqrt_nr(x):
    i = plsc.bitcast(x, jnp.int32)
    i = jnp.int32(0x5F3759DF) - lax.shift_right_logical(i, 1)
    y = plsc.bitcast(i, jnp.float32)
    xh = x * jnp.float32(0.5)
    for _ in range(3):
        y = y * (jnp.float32(1.5) - xh * y * y)
    return y


def _sc_transe(node_hbm, rel_hbm, hidx_hbm, ridx_hbm, tidx_hbm, out_hbm,
               hiv, riv, tiv, hrows, rrows, trows,
               outv, s1, s2, s3):
    wid = lax.axis_index("s") * NC + lax.axis_index("c")
    base = wid * BPW

    pltpu.sync_copy(hidx_hbm.at[pl.ds(base, BPW)], hiv)
    pltpu.sync_copy(tidx_hbm.at[pl.ds(base, BPW)], tiv)
    pltpu.sync_copy(ridx_hbm.at[pl.ds(base, BPW)], riv)

    c1 = pltpu.async_copy(node_hbm.at[hiv], hrows, s1)
    c2 = pltpu.async_copy(node_hbm.at[tiv], trows, s2)
    c3 = pltpu.async_copy(rel_hbm.at[riv], rrows, s3)
    c1.wait()
    c2.wait()
    c3.wait()

    iota = lax.iota(jnp.int32, L)
    zero = jnp.zeros((L,), jnp.float32)

    def block(b, _):
        hh = tt = rr = hr = ht = rt = zero
        for i in range(L):
            ri = b * L + i
            h0 = hrows[ri, pl.ds(0, L)]
            h1 = hrows[ri, pl.ds(L, L)]
            t0 = trows[ri, pl.ds(0, L)]
            t1 = trows[ri, pl.ds(L, L)]
            r0 = rrows[ri, pl.ds(0, L)]
            r1 = rrows[ri, pl.ds(L, L)]
            lane = iota == i
            hh = jnp.where(lane, jnp.sum(h0 * h0 + h1 * h1), hh)
            tt = jnp.where(lane, jnp.sum(t0 * t0 + t1 * t1), tt)
            rr = jnp.where(lane, jnp.sum(r0 * r0 + r1 * r1), rr)
            hr = jnp.where(lane, jnp.sum(h0 * r0 + h1 * r1), hr)
            ht = jnp.where(lane, jnp.sum(h0 * t0 + h1 * t1), ht)
            rt = jnp.where(lane, jnp.sum(r0 * t0 + r1 * t1), rt)
        irh = _rsqrt_nr(jnp.maximum(hh, jnp.float32(1e-24)))
        irt = _rsqrt_nr(jnp.maximum(tt, jnp.float32(1e-24)))
        aa = hh * irh * irh
        bb = tt * irt * irt
        cross = hr * irh - ht * (irh * irt) - rt * irt
        dd = aa + bb + rr + (cross + cross)
        s = jnp.maximum(dd, jnp.float32(0.0))
        outv[pl.ds(b * L, L)] = -(s * _rsqrt_nr(jnp.maximum(s, jnp.float32(1e-30))))
        return _

    lax.fori_loop(0, BLOCKS, block, None)

    pltpu.sync_copy(outv, out_hbm.at[pl.ds(base, BPW)])


@jax.jit
def _transe_sc(node_emb, rel_emb, hidx, ridx, tidx):
    mesh = plsc.VectorSubcoreMesh(
        core_axis_name="c", subcore_axis_name="s",
        num_cores=NC, num_subcores=NS)
    f = pl.kernel(
        _sc_transe,
        out_type=jax.ShapeDtypeStruct((B,), jnp.float32),
        mesh=mesh,
        compiler_params=pltpu.CompilerParams(
            needs_layout_passes=False, use_tc_tiling_on_sc=False),
        scratch_types=[
            pltpu.VMEM((BPW,), jnp.int32),
            pltpu.VMEM((BPW,), jnp.int32),
            pltpu.VMEM((BPW,), jnp.int32),
            pltpu.VMEM((BPW, D), jnp.float32),
            pltpu.VMEM((BPW, D), jnp.float32),
            pltpu.VMEM((BPW, D), jnp.float32),
            pltpu.VMEM((BPW,), jnp.float32),
            pltpu.SemaphoreType.DMA,
            pltpu.SemaphoreType.DMA,
            pltpu.SemaphoreType.DMA,
        ],
    )
    return f(node_emb, rel_emb, hidx, ridx, tidx)


def kernel(head_index, rel_type, tail_index, node_emb, rel_emb):
    hidx = head_index.astype(jnp.int32)
    ridx = rel_type.astype(jnp.int32)
    tidx = tail_index.astype(jnp.int32)
    return _transe_sc(node_emb, rel_emb, hidx, ridx, tidx)


# speedup vs baseline: 1.0946x; 1.0002x over previous
import jax
import jax.numpy as jnp
from jax import lax
from jax.experimental import pallas as pl
from jax.experimental.pallas import tpu as pltpu
from jax.experimental.pallas import tpu_sc as plsc

NC = 2
NS = 16
L = 16
NW = NC * NS

B = 16384
D = 32
BPW = B // NW
BLOCKS = BPW // L


def _rsqrt_nr(x):
    i = plsc.bitcast(x, jnp.int32)
    i = jnp.int32(0x5F3759DF) - lax.shift_right_logical(i, 1)
    y = plsc.bitcast(i, jnp.float32)
    xh = x * jnp.float32(0.5)
    for _ in range(3):
        y = y * (jnp.float32(1.5) - xh * y * y)
    return y


def _sc_transe(node_hbm, rel_hbm, hidx_hbm, ridx_hbm, tidx_hbm, out_hbm,
               hiv, riv, tiv, hrows, rrows, trows,
               outv, s1, s2, s3):
    wid = lax.axis_index("s") * NC + lax.axis_index("c")
    base = wid * BPW

    pltpu.sync_copy(hidx_hbm.at[pl.ds(base, BPW)], hiv)
    pltpu.sync_copy(tidx_hbm.at[pl.ds(base, BPW)], tiv)
    pltpu.sync_copy(ridx_hbm.at[pl.ds(base, BPW)], riv)

    c1 = pltpu.async_copy(node_hbm.at[hiv], hrows, s1)
    c2 = pltpu.async_copy(node_hbm.at[tiv], trows, s2)
    c3 = pltpu.async_copy(rel_hbm.at[riv], rrows, s3)
    c1.wait()
    c2.wait()
    c3.wait()

    iota = lax.iota(jnp.int32, L)
    zero = jnp.zeros((L,), jnp.float32)

    def block(b, _):
        hh = tt = rr = hr = ht = rt = zero
        for i in range(L):
            ri = b * L + i
            h0 = hrows[ri, pl.ds(0, L)]
            h1 = hrows[ri, pl.ds(L, L)]
            t0 = trows[ri, pl.ds(0, L)]
            t1 = trows[ri, pl.ds(L, L)]
            r0 = rrows[ri, pl.ds(0, L)]
            r1 = rrows[ri, pl.ds(L, L)]
            lane = iota == i
            hh = jnp.where(lane, jnp.sum(h0 * h0 + h1 * h1), hh)
            tt = jnp.where(lane, jnp.sum(t0 * t0 + t1 * t1), tt)
            rr = jnp.where(lane, jnp.sum(r0 * r0 + r1 * r1), rr)
            hr = jnp.where(lane, jnp.sum(h0 * r0 + h1 * r1), hr)
            ht = jnp.where(lane, jnp.sum(h0 * t0 + h1 * t1), ht)
            rt = jnp.where(lane, jnp.sum(r0 * t0 + r1 * t1), rt)
        irh = _rsqrt_nr(jnp.maximum(hh, jnp.float32(1e-24)))
        irt = _rsqrt_nr(jnp.maximum(tt, jnp.float32(1e-24)))
        aa = hh * irh * irh
        bb = tt * irt * irt
        cross = hr * irh - ht * (irh * irt) - rt * irt
        dd = aa + bb + rr + (cross + cross)
        s = jnp.maximum(dd, jnp.float32(0.0))
        outv[pl.ds(b * L, L)] = -(s * _rsqrt_nr(jnp.maximum(s, jnp.float32(1e-30))))
        return _

    lax.fori_loop(0, BLOCKS, block, None)

    pltpu.sync_copy(outv, out_hbm.at[pl.ds(base, BPW)])


@jax.jit
def _transe_sc(node_emb, rel_emb, hidx, ridx, tidx):
    mesh = plsc.VectorSubcoreMesh(
        core_axis_name="c", subcore_axis_name="s",
        num_cores=NC, num_subcores=NS)
    f = pl.kernel(
        _sc_transe,
        out_type=jax.ShapeDtypeStruct((B,), jnp.float32),
        mesh=mesh,
        compiler_params=pltpu.CompilerParams(
            needs_layout_passes=False, use_tc_tiling_on_sc=False,
            skip_device_barrier=True),
        scratch_types=[
            pltpu.VMEM((BPW,), jnp.int32),
            pltpu.VMEM((BPW,), jnp.int32),
            pltpu.VMEM((BPW,), jnp.int32),
            pltpu.VMEM((BPW, D), jnp.float32),
            pltpu.VMEM((BPW, D), jnp.float32),
            pltpu.VMEM((BPW, D), jnp.float32),
            pltpu.VMEM((BPW,), jnp.float32),
            pltpu.SemaphoreType.DMA,
            pltpu.SemaphoreType.DMA,
            pltpu.SemaphoreType.DMA,
        ],
    )
    return f(node_emb, rel_emb, hidx, ridx, tidx)


def kernel(head_index, rel_type, tail_index, node_emb, rel_emb):
    hidx = head_index.astype(jnp.int32)
    ridx = rel_type.astype(jnp.int32)
    tidx = tail_index.astype(jnp.int32)
    return _transe_sc(node_emb, rel_emb, hidx, ridx, tidx)
